# Initial kernel scaffold; baseline (speedup 1.0000x reference)
import jax
import jax.numpy as jnp
from jax.experimental import pallas as pl


def kernel(x, angles, bias):
    raise NotImplementedError("write your pallas kernel here")


# speedup vs baseline: 6.5298x; 6.5298x over previous
import functools

import jax
import jax.numpy as jnp
import numpy as np
from jax import lax
from jax.experimental import pallas as pl
from jax.experimental.pallas import tpu as pltpu


@functools.lru_cache(maxsize=None)
def _schedule_arrays(d: int):
    pairs = [(i, j) for i in range(d) for j in range(i + 1, d)]
    pair_to_idx = {pair: idx for idx, pair in enumerate(pairs)}
    schedule = []
    remaining_pairs = set(pairs)
    while remaining_pairs:
        step = []
        used = set()
        for pair in list(remaining_pairs):
            i, j = pair
            if i not in used and j not in used:
                step.append(pair)
                used.update([i, j])
        schedule.append(step)
        remaining_pairs -= set(step)

    S = len(schedule)
    S_pad = ((S + 7) // 8) * 8
    perm = np.tile(np.arange(d, dtype=np.int32), (S_pad, 1))
    aidx = np.zeros((S_pad, d), dtype=np.int32)
    sign = np.zeros((S_pad, d), dtype=np.float32)
    for t, step in enumerate(schedule):
        for (i, j) in step:
            a = pair_to_idx[(i, j)]
            perm[t, i] = j
            perm[t, j] = i
            aidx[t, i] = a
            aidx[t, j] = a
            sign[t, i] = -1.0
            sign[t, j] = 1.0
    return perm, aidx, sign, S_pad


def _givens_body(x_ref, ang_ref, perm_ref, bias_ref, out_ref, m_ref, *, S, d):
    @pl.when(pl.program_id(0) == 0)
    def _build_m():
        rows = lax.broadcasted_iota(jnp.int32, (d, d), 0)
        cols = lax.broadcasted_iota(jnp.int32, (d, d), 1)
        eye = rows == cols
        m_ref[:, :] = jnp.where(eye, jnp.float32(1.0), jnp.float32(0.0))

        def body(t, _):
            a = ang_ref[pl.ds(t, 1), :]
            p = perm_ref[pl.ds(t, 1), :]
            c = jnp.cos(a)
            s = jnp.sin(a)
            r = jnp.where(eye, c, jnp.float32(0.0)) + jnp.where(
                rows == p, s, jnp.float32(0.0))
            m_ref[:, :] = jnp.dot(m_ref[:, :], r,
                                  preferred_element_type=jnp.float32,
                                  precision=lax.Precision.HIGHEST)
            return 0

        lax.fori_loop(0, S, body, 0)

    out_ref[:, :] = (
        jnp.dot(x_ref[:, :], m_ref[:, :], preferred_element_type=jnp.float32,
                precision=lax.Precision.HIGHEST)
        + bias_ref[:, :]
    )


def kernel(x, angles, bias):
    B, d = x.shape
    perm_np, aidx_np, sign_np, S = _schedule_arrays(d)
    ang = (angles[aidx_np] * sign_np).astype(jnp.float32)
    perm = jnp.asarray(perm_np)

    block = 512
    grid = B // block
    return pl.pallas_call(
        functools.partial(_givens_body, S=S, d=d),
        grid=(grid,),
        in_specs=[
            pl.BlockSpec((block, d), lambda i: (i, 0)),
            pl.BlockSpec((S, d), lambda i: (0, 0)),
            pl.BlockSpec((S, d), lambda i: (0, 0)),
            pl.BlockSpec((1, d), lambda i: (0, 0)),
        ],
        out_specs=pl.BlockSpec((block, d), lambda i: (i, 0)),
        out_shape=jax.ShapeDtypeStruct((B, d), jnp.float32),
        scratch_shapes=[pltpu.VMEM((d, d), jnp.float32)],
    )(x, ang, perm, bias.reshape(1, d))


# speedup vs baseline: 7.1066x; 1.0883x over previous
import functools

import jax
import jax.numpy as jnp
import numpy as np
from jax import lax
from jax.experimental import pallas as pl
from jax.experimental.pallas import tpu as pltpu

_K = 8


@functools.lru_cache(maxsize=None)
def _schedule_arrays(d: int, k_chains: int):
    pairs = [(i, j) for i in range(d) for j in range(i + 1, d)]
    pair_to_idx = {pair: idx for idx, pair in enumerate(pairs)}
    schedule = []
    remaining_pairs = set(pairs)
    while remaining_pairs:
        step = []
        used = set()
        for pair in list(remaining_pairs):
            i, j = pair
            if i not in used and j not in used:
                step.append(pair)
                used.update([i, j])
        schedule.append(step)
        remaining_pairs -= set(step)

    S = len(schedule)
    L = -(-S // k_chains)
    S_pad = k_chains * L
    perm = np.tile(np.arange(d, dtype=np.int32), (S_pad, 1))
    aidx = np.zeros((S_pad, d), dtype=np.int32)
    sign = np.zeros((S_pad, d), dtype=np.float32)
    for t, step in enumerate(schedule):
        for (i, j) in step:
            a = pair_to_idx[(i, j)]
            perm[t, i] = j
            perm[t, j] = i
            aidx[t, i] = a
            aidx[t, j] = a
            sign[t, i] = -1.0
            sign[t, j] = 1.0
    return perm, aidx, sign, L


def _givens_body(x_ref, ang_ref, perm_ref, bias_ref, out_ref, p_ref, m_ref,
                 *, L, K, d):
    rows = lax.broadcasted_iota(jnp.int32, (d, d), 0)
    cols = lax.broadcasted_iota(jnp.int32, (d, d), 1)
    eye = rows == cols
    zero = jnp.float32(0.0)

    def build_r(t):
        a = ang_ref[pl.ds(t, 1), :]
        p = perm_ref[pl.ds(t, 1), :]
        c = jnp.cos(a)
        s = jnp.sin(a)
        return jnp.where(eye, c, zero) + jnp.where(rows == p, s, zero)

    def mm(a, b):
        return jnp.dot(a, b, preferred_element_type=jnp.float32,
                       precision=lax.Precision.HIGHEST)

    @pl.when(pl.program_id(0) == 0)
    def _build_m():
        for c in range(K):
            p_ref[c] = build_r(c * L)

        def body(t, _):
            for c in range(K):
                p_ref[c] = mm(p_ref[c], build_r(c * L + t))
            return 0

        lax.fori_loop(1, L, body, 0)

        q0 = mm(p_ref[0], p_ref[1])
        q1 = mm(p_ref[2], p_ref[3])
        q2 = mm(p_ref[4], p_ref[5])
        q3 = mm(p_ref[6], p_ref[7])
        m_ref[:, :] = mm(mm(q0, q1), mm(q2, q3))

    out_ref[:, :] = (
        jnp.dot(x_ref[:, :], m_ref[:, :], preferred_element_type=jnp.float32,
                precision=lax.Precision.HIGHEST)
        + bias_ref[:, :]
    )


def kernel(x, angles, bias):
    B, d = x.shape
    perm_np, aidx_np, sign_np, L = _schedule_arrays(d, _K)
    ang = (angles[aidx_np] * sign_np).astype(jnp.float32)
    perm = jnp.asarray(perm_np)
    S = _K * L

    block = 512
    grid = B // block
    return pl.pallas_call(
        functools.partial(_givens_body, L=L, K=_K, d=d),
        grid=(grid,),
        in_specs=[
            pl.BlockSpec((block, d), lambda i: (i, 0)),
            pl.BlockSpec((S, d), lambda i: (0, 0)),
            pl.BlockSpec((S, d), lambda i: (0, 0)),
            pl.BlockSpec((1, d), lambda i: (0, 0)),
        ],
        out_specs=pl.BlockSpec((block, d), lambda i: (i, 0)),
        out_shape=jax.ShapeDtypeStruct((B, d), jnp.float32),
        scratch_shapes=[pltpu.VMEM((_K, d, d), jnp.float32),
                        pltpu.VMEM((d, d), jnp.float32)],
    )(x, ang, perm, bias.reshape(1, d))


# speedup vs baseline: 17.3997x; 2.4484x over previous
import functools

import jax
import jax.numpy as jnp
import numpy as np
from jax import lax
from jax.experimental import pallas as pl
from jax.experimental.pallas import tpu as pltpu
from jax.experimental.pallas import tpu_sc as plsc

_NSEG = 32


@functools.lru_cache(maxsize=None)
def _flat_ops(d: int):
    pairs = [(i, j) for i in range(d) for j in range(i + 1, d)]
    pair_to_idx = {pair: idx for idx, pair in enumerate(pairs)}
    schedule = []
    remaining_pairs = set(pairs)
    while remaining_pairs:
        step = []
        used = set()
        for pair in list(remaining_pairs):
            i, j = pair
            if i not in used and j not in used:
                step.append(pair)
                used.update([i, j])
        schedule.append(step)
        remaining_pairs -= set(step)

    ops = [(i, j, pair_to_idx[(i, j)]) for st in schedule for (i, j) in st]
    n = len(ops)
    chunk = -(-n // _NSEG)
    per = ((chunk + 7) // 8) * 8
    kp = np.zeros((_NSEG, per), dtype=np.int32)
    aidx = np.zeros((_NSEG, per), dtype=np.int32)
    valid = np.zeros((_NSEG, per), dtype=np.float32)
    for o, (i, j, a) in enumerate(ops):
        g = o // chunk
        p = o - g * chunk
        kp[g, p] = i * d + j
        aidx[g, p] = a
        valid[g, p] = 1.0
    kp = kp[:, ::-1].copy()
    aidx = aidx[:, ::-1].copy()
    valid = valid[:, ::-1].copy()
    return kp, aidx, valid, per


def _trig_body(a_ref, c_ref, s_ref):
    a = a_ref[...]
    c_ref[...] = jnp.cos(a)
    s_ref[...] = jnp.sin(a)


def _sc_build_body(c_hbm, s_hbm, kp_hbm, eye_hbm, out_hbm,
                   c_v, s_v, kp_v, slab_v, *, n_op, d):
    wid = lax.axis_index("s") * 2 + lax.axis_index("c")
    pltpu.sync_copy(c_hbm.at[wid], c_v)
    pltpu.sync_copy(s_hbm.at[wid], s_v)
    pltpu.sync_copy(kp_hbm.at[wid], kp_v)
    pltpu.sync_copy(eye_hbm, slab_v)

    def body(b, carry):
        kpv = kp_v[pl.ds(b * 16, 16)]
        cv = c_v[pl.ds(b * 16, 16)]
        sv = s_v[pl.ds(b * 16, 16)]
        for u in range(16):
            kp = kpv[u]
            pk = jnp.bitwise_and(kp, d - 1)
            kbase = kp - pk
            pkbase = pk * d
            c = cv[u]
            s = sv[u]
            for r in range(d // 16):
                o = r * 16
                vk = slab_v[pl.ds(kbase + o, 16)]
                vp = slab_v[pl.ds(pkbase + o, 16)]
                slab_v[pl.ds(kbase + o, 16)] = c * vk - s * vp
                slab_v[pl.ds(pkbase + o, 16)] = s * vk + c * vp
        return carry

    lax.fori_loop(0, n_op // 16, body, 0)
    pltpu.sync_copy(slab_v, out_hbm.at[wid])


def _combine_body(x_ref, p_ref, bias_ref, out_ref, q_ref, m_ref, *, d):
    def mm(a, b):
        return jnp.dot(a, b, preferred_element_type=jnp.float32,
                       precision=lax.Precision.HIGHEST)

    @pl.when(pl.program_id(0) == 0)
    def _combine():
        for i in range(16):
            q_ref[i] = mm(p_ref[2 * i], p_ref[2 * i + 1])
        for i in range(8):
            q_ref[i] = mm(q_ref[2 * i], q_ref[2 * i + 1])
        for i in range(4):
            q_ref[i] = mm(q_ref[2 * i], q_ref[2 * i + 1])
        q_ref[0] = mm(q_ref[0], q_ref[1])
        q_ref[1] = mm(q_ref[2], q_ref[3])
        m_ref[:, :] = mm(q_ref[0], q_ref[1])

    out_ref[:, :] = (
        jnp.dot(x_ref[:, :], m_ref[:, :], preferred_element_type=jnp.float32,
                precision=lax.Precision.HIGHEST)
        + bias_ref[:, :]
    )


def kernel(x, angles, bias):
    B, d = x.shape
    kp_np, aidx_np, valid_np, n_op = _flat_ops(d)

    a_op = (angles[aidx_np] * (-valid_np)).astype(jnp.float32)

    cs_shape = jax.ShapeDtypeStruct((_NSEG, n_op), jnp.float32)
    c3, s3 = pl.pallas_call(
        _trig_body, out_shape=(cs_shape, cs_shape))(a_op)

    sc_build = pl.kernel(
        functools.partial(_sc_build_body, n_op=n_op, d=d),
        out_type=jax.ShapeDtypeStruct((_NSEG, d * d), jnp.float32),
        mesh=plsc.VectorSubcoreMesh(core_axis_name="c", subcore_axis_name="s"),
        scratch_types=[
            pltpu.VMEM((n_op,), jnp.float32),
            pltpu.VMEM((n_op,), jnp.float32),
            pltpu.VMEM((n_op,), jnp.int32),
            pltpu.VMEM((d * d,), jnp.float32),
        ],
    )
    eye = jnp.eye(d, dtype=jnp.float32).reshape(d * d)
    partials = sc_build(c3, s3, jnp.asarray(kp_np), eye)
    partials = partials.reshape(_NSEG, d, d)

    block = 512
    grid = B // block
    return pl.pallas_call(
        functools.partial(_combine_body, d=d),
        grid=(grid,),
        in_specs=[
            pl.BlockSpec((block, d), lambda i: (i, 0)),
            pl.BlockSpec((_NSEG, d, d), lambda i: (0, 0, 0)),
            pl.BlockSpec((1, d), lambda i: (0, 0)),
        ],
        out_specs=pl.BlockSpec((block, d), lambda i: (i, 0)),
        out_shape=jax.ShapeDtypeStruct((B, d), jnp.float32),
        scratch_shapes=[pltpu.VMEM((16, d, d), jnp.float32),
                        pltpu.VMEM((d, d), jnp.float32)],
    )(x, partials, bias.reshape(1, d))


# speedup vs baseline: 32.3243x; 1.8578x over previous
import functools

import jax
import jax.numpy as jnp
import numpy as np
from jax import lax
from jax.experimental import pallas as pl
from jax.experimental.pallas import tpu as pltpu
from jax.experimental.pallas import tpu_sc as plsc

_NSEG = 32


@functools.lru_cache(maxsize=None)
def _flat_ops(d: int):
    pairs = [(i, j) for i in range(d) for j in range(i + 1, d)]
    pair_to_idx = {pair: idx for idx, pair in enumerate(pairs)}
    schedule = []
    remaining_pairs = set(pairs)
    while remaining_pairs:
        step = []
        used = set()
        for pair in list(remaining_pairs):
            i, j = pair
            if i not in used and j not in used:
                step.append(pair)
                used.update([i, j])
        schedule.append(step)
        remaining_pairs -= set(step)

    ops = [(i, j, pair_to_idx[(i, j)]) for st in schedule for (i, j) in st]
    n = len(ops)
    chunk = -(-n // _NSEG)
    per = ((chunk + 7) // 8) * 8
    kp = np.zeros((_NSEG, per), dtype=np.int32)
    aidx = np.zeros((_NSEG, per), dtype=np.int32)
    valid = np.zeros((_NSEG, per), dtype=np.float32)
    for o, (i, j, a) in enumerate(ops):
        g = o // chunk
        p = o - g * chunk
        kp[g, p] = i * d + j
        aidx[g, p] = a
        valid[g, p] = 1.0
    kp = kp[:, ::-1].copy()
    aidx = aidx[:, ::-1].copy()
    valid = valid[:, ::-1].copy()
    return kp, aidx, valid, per


def _trig_body(a_ref, c_ref, s_ref):
    a = a_ref[...]
    c_ref[...] = jnp.cos(a)
    s_ref[...] = jnp.sin(a)


def _sc_build_body(c_hbm, s_hbm, kp_hbm, eye_hbm, out_hbm,
                   c_v, s_v, kp_v, slab_v, *, n_op, d):
    wid = lax.axis_index("s") * 2 + lax.axis_index("c")
    pltpu.sync_copy(c_hbm.at[wid], c_v)
    pltpu.sync_copy(s_hbm.at[wid], s_v)
    pltpu.sync_copy(kp_hbm.at[wid], kp_v)
    pltpu.sync_copy(eye_hbm, slab_v)

    nr = d // 16

    def body(b, carry):
        kpv = kp_v[pl.ds(b * 16, 16)]
        for u in range(16):
            kp = kpv[u]
            pk = jnp.bitwise_and(kp, d - 1)
            kbase = kp - pk
            pkbase = pk * d
            coff = b * 256 + u * 16
            c = c_v[pl.ds(coff, 16)]
            s = s_v[pl.ds(coff, 16)]
            vks = [slab_v[pl.ds(kbase + 16 * r, 16)] for r in range(nr)]
            vps = [slab_v[pl.ds(pkbase + 16 * r, 16)] for r in range(nr)]
            for r in range(nr):
                slab_v[pl.ds(kbase + 16 * r, 16)] = c * vks[r] - s * vps[r]
            for r in range(nr):
                slab_v[pl.ds(pkbase + 16 * r, 16)] = s * vks[r] + c * vps[r]
        return carry

    lax.fori_loop(0, n_op // 16, body, 0)
    pltpu.sync_copy(slab_v, out_hbm.at[wid])


def _combine_body(x_ref, p_ref, bias_ref, out_ref, q_ref, m_ref, *, d):
    def mm(a, b):
        return jnp.dot(a, b, preferred_element_type=jnp.float32,
                       precision=lax.Precision.HIGHEST)

    @pl.when(pl.program_id(0) == 0)
    def _combine():
        for i in range(16):
            q_ref[i] = mm(p_ref[2 * i], p_ref[2 * i + 1])
        for i in range(8):
            q_ref[i] = mm(q_ref[2 * i], q_ref[2 * i + 1])
        for i in range(4):
            q_ref[i] = mm(q_ref[2 * i], q_ref[2 * i + 1])
        q_ref[0] = mm(q_ref[0], q_ref[1])
        q_ref[1] = mm(q_ref[2], q_ref[3])
        m_ref[:, :] = mm(q_ref[0], q_ref[1])

    out_ref[:, :] = (
        jnp.dot(x_ref[:, :], m_ref[:, :], preferred_element_type=jnp.float32,
                precision=lax.Precision.HIGHEST)
        + bias_ref[:, :]
    )


def kernel(x, angles, bias):
    B, d = x.shape
    kp_np, aidx_np, valid_np, n_op = _flat_ops(d)

    a_op = (angles[aidx_np] * (-valid_np)).astype(jnp.float32)

    cs_shape = jax.ShapeDtypeStruct((_NSEG, n_op), jnp.float32)
    c3, s3 = pl.pallas_call(
        _trig_body, out_shape=(cs_shape, cs_shape))(a_op)
    c3 = jnp.broadcast_to(c3[:, :, None], (_NSEG, n_op, 16)).reshape(
        _NSEG, n_op * 16)
    s3 = jnp.broadcast_to(s3[:, :, None], (_NSEG, n_op, 16)).reshape(
        _NSEG, n_op * 16)

    sc_build = pl.kernel(
        functools.partial(_sc_build_body, n_op=n_op, d=d),
        out_type=jax.ShapeDtypeStruct((_NSEG, d * d), jnp.float32),
        mesh=plsc.VectorSubcoreMesh(core_axis_name="c", subcore_axis_name="s"),
        scratch_types=[
            pltpu.VMEM((n_op * 16,), jnp.float32),
            pltpu.VMEM((n_op * 16,), jnp.float32),
            pltpu.VMEM((n_op,), jnp.int32),
            pltpu.VMEM((d * d,), jnp.float32),
        ],
    )
    eye = jnp.eye(d, dtype=jnp.float32).reshape(d * d)
    partials = sc_build(c3, s3, jnp.asarray(kp_np), eye)
    partials = partials.reshape(_NSEG, d, d)

    block = 512
    grid = B // block
    return pl.pallas_call(
        functools.partial(_combine_body, d=d),
        grid=(grid,),
        in_specs=[
            pl.BlockSpec((block, d), lambda i: (i, 0)),
            pl.BlockSpec((_NSEG, d, d), lambda i: (0, 0, 0)),
            pl.BlockSpec((1, d), lambda i: (0, 0)),
        ],
        out_specs=pl.BlockSpec((block, d), lambda i: (i, 0)),
        out_shape=jax.ShapeDtypeStruct((B, d), jnp.float32),
        scratch_shapes=[pltpu.VMEM((16, d, d), jnp.float32),
                        pltpu.VMEM((d, d), jnp.float32)],
    )(x, partials, bias.reshape(1, d))


# speedup vs baseline: 35.8282x; 1.1084x over previous
import functools

import jax
import jax.numpy as jnp
import numpy as np
from jax import lax
from jax.experimental import pallas as pl
from jax.experimental.pallas import tpu as pltpu
from jax.experimental.pallas import tpu_sc as plsc

_NSEG = 32


@functools.lru_cache(maxsize=None)
def _flat_ops(d: int):
    pairs = [(i, j) for i in range(d) for j in range(i + 1, d)]
    pair_to_idx = {pair: idx for idx, pair in enumerate(pairs)}
    schedule = []
    remaining_pairs = set(pairs)
    while remaining_pairs:
        step = []
        used = set()
        for pair in list(remaining_pairs):
            i, j = pair
            if i not in used and j not in used:
                step.append(pair)
                used.update([i, j])
        schedule.append(step)
        remaining_pairs -= set(step)

    ops = [(i, j, pair_to_idx[(i, j)], t)
           for t, st in enumerate(schedule) for (i, j) in st]
    n = len(ops)
    chunk = -(-n // _NSEG)
    per = ((chunk + 7) // 8) * 8
    kp = np.zeros((_NSEG, per), dtype=np.int32)
    aidx = np.zeros((_NSEG, per), dtype=np.int32)
    valid = np.zeros((_NSEG, per), dtype=np.float32)
    sid = np.full((_NSEG, per), -1, dtype=np.int32)
    for g in range(_NSEG):
        for p in range(chunk, per):
            q = p - chunk
            kp[g, p] = (2 * q) * d + (2 * q + 1)
    for o, (i, j, a, t) in enumerate(ops):
        g = o // chunk
        p = o - g * chunk
        kp[g, p] = i * d + j
        aidx[g, p] = a
        valid[g, p] = 1.0
        sid[g, p] = t
    kp = kp[:, ::-1].copy()
    aidx = aidx[:, ::-1].copy()
    valid = valid[:, ::-1].copy()
    sid = sid[:, ::-1].copy()
    nb = 16
    bnd = np.full((_NSEG, nb), per, dtype=np.int32)
    for g in range(_NSEG):
        bounds = [0]
        for p in range(1, per):
            if sid[g, p] != sid[g, p - 1]:
                bounds.append(p)
        bounds.append(per)
        assert len(bounds) <= nb, len(bounds)
        bnd[g, :len(bounds)] = np.asarray(bounds, dtype=np.int32)
    return kp, aidx, valid, bnd, per


def _trig_body(a_ref, c_ref, s_ref):
    a = a_ref[...]
    c_ref[...] = jnp.cos(a)
    s_ref[...] = jnp.sin(a)


def _sc_build_body(c_hbm, s_hbm, kp_hbm, bnd_hbm, eye_hbm, out_hbm,
                   c_v, s_v, kp_v, bnd_v, slab_v, kp_sm, *, n_op, d):
    wid = lax.axis_index("s") * 2 + lax.axis_index("c")
    pltpu.sync_copy(c_hbm.at[wid], c_v)
    pltpu.sync_copy(s_hbm.at[wid], s_v)
    pltpu.sync_copy(kp_hbm.at[wid], kp_v)
    pltpu.sync_copy(bnd_hbm.at[wid], bnd_v)
    pltpu.sync_copy(eye_hbm, slab_v)

    nr = d // 16

    def stage(b, carry):
        kpv = kp_v[pl.ds(b * 16, 16)]
        for u in range(16):
            kp_sm[b * 16 + u] = kpv[u]
        return carry

    lax.fori_loop(0, n_op // 16, stage, 0)

    def op_body(i):
        kp = kp_sm[i]
        pk = jnp.bitwise_and(kp, d - 1)
        kbase = kp - pk
        pkbase = pk * d
        c = c_v[pl.ds(i * 16, 16)]
        s = s_v[pl.ds(i * 16, 16)]
        vks = [slab_v[pl.ds(kbase + 16 * r, 16)] for r in range(nr)]
        vps = [slab_v[pl.ds(pkbase + 16 * r, 16)] for r in range(nr)]
        for r in range(nr):
            slab_v[pl.ds(kbase + 16 * r, 16)] = c * vks[r] - s * vps[r]
        for r in range(nr):
            slab_v[pl.ds(pkbase + 16 * r, 16)] = s * vks[r] + c * vps[r]

    bndv = bnd_v[pl.ds(0, 16)]
    for j in range(15):
        lo = bndv[j]
        hi = bndv[j + 1]

        @plsc.parallel_loop(lo, hi, 1, unroll=1)
        def _(i):
            op_body(i)

    pltpu.sync_copy(slab_v, out_hbm.at[wid])


def _combine_body(x_ref, p_ref, bias_ref, out_ref, q_ref, m_ref, *, d):
    def mm(a, b):
        return jnp.dot(a, b, preferred_element_type=jnp.float32,
                       precision=lax.Precision.HIGHEST)

    @pl.when(pl.program_id(0) == 0)
    def _combine():
        for i in range(16):
            q_ref[i] = mm(p_ref[2 * i], p_ref[2 * i + 1])
        for i in range(8):
            q_ref[i] = mm(q_ref[2 * i], q_ref[2 * i + 1])
        for i in range(4):
            q_ref[i] = mm(q_ref[2 * i], q_ref[2 * i + 1])
        q_ref[0] = mm(q_ref[0], q_ref[1])
        q_ref[1] = mm(q_ref[2], q_ref[3])
        m_ref[:, :] = mm(q_ref[0], q_ref[1])

    out_ref[:, :] = (
        jnp.dot(x_ref[:, :], m_ref[:, :], preferred_element_type=jnp.float32,
                precision=lax.Precision.HIGHEST)
        + bias_ref[:, :]
    )


def kernel(x, angles, bias):
    B, d = x.shape
    kp_np, aidx_np, valid_np, bnd_np, n_op = _flat_ops(d)

    a_op = (angles[aidx_np] * (-valid_np)).astype(jnp.float32)

    cs_shape = jax.ShapeDtypeStruct((_NSEG, n_op), jnp.float32)
    c3, s3 = pl.pallas_call(
        _trig_body, out_shape=(cs_shape, cs_shape))(a_op)
    c3 = jnp.broadcast_to(c3[:, :, None], (_NSEG, n_op, 16)).reshape(
        _NSEG, n_op * 16)
    s3 = jnp.broadcast_to(s3[:, :, None], (_NSEG, n_op, 16)).reshape(
        _NSEG, n_op * 16)

    sc_build = pl.kernel(
        functools.partial(_sc_build_body, n_op=n_op, d=d),
        out_type=jax.ShapeDtypeStruct((_NSEG, d * d), jnp.float32),
        mesh=plsc.VectorSubcoreMesh(core_axis_name="c", subcore_axis_name="s"),
        scratch_types=[
            pltpu.VMEM((n_op * 16,), jnp.float32),
            pltpu.VMEM((n_op * 16,), jnp.float32),
            pltpu.VMEM((n_op,), jnp.int32),
            pltpu.VMEM((16,), jnp.int32),
            pltpu.VMEM((d * d,), jnp.float32),
            pltpu.SMEM((n_op,), jnp.int32),
        ],
    )
    eye = jnp.eye(d, dtype=jnp.float32).reshape(d * d)
    partials = sc_build(c3, s3, jnp.asarray(kp_np), jnp.asarray(bnd_np), eye)
    partials = partials.reshape(_NSEG, d, d)

    block = 512
    grid = B // block
    return pl.pallas_call(
        functools.partial(_combine_body, d=d),
        grid=(grid,),
        in_specs=[
            pl.BlockSpec((block, d), lambda i: (i, 0)),
            pl.BlockSpec((_NSEG, d, d), lambda i: (0, 0, 0)),
            pl.BlockSpec((1, d), lambda i: (0, 0)),
        ],
        out_specs=pl.BlockSpec((block, d), lambda i: (i, 0)),
        out_shape=jax.ShapeDtypeStruct((B, d), jnp.float32),
        scratch_shapes=[pltpu.VMEM((16, d, d), jnp.float32),
                        pltpu.VMEM((d, d), jnp.float32)],
    )(x, partials, bias.reshape(1, d))
